# Initial kernel scaffold; baseline (speedup 1.0000x reference)
import jax
import jax.numpy as jnp
from jax.experimental import pallas as pl


def kernel(sequence):
    raise NotImplementedError("write your pallas kernel here")


# speedup vs baseline: 7.6524x; 7.6524x over previous
import jax
import jax.numpy as jnp
from jax.experimental import pallas as pl

_ALPHA = 1000
_SEQ = 16384


def kernel(sequence):
    C = 2048

    def body(seq_ref, out_ref):
        seq = seq_ref[...].astype(jnp.int32)
        rows = jax.lax.broadcasted_iota(jnp.int32, (_ALPHA, C), 0)
        out_ref[...] = (rows == seq[None, :]).astype(jnp.float32)

    return pl.pallas_call(
        body,
        grid=(_SEQ // C,),
        in_specs=[pl.BlockSpec((C,), lambda i: (i,))],
        out_specs=pl.BlockSpec((_ALPHA, C), lambda i: (0, i)),
        out_shape=jax.ShapeDtypeStruct((_ALPHA, _SEQ), jnp.float32),
    )(sequence)
